# baseline (speedup 1.0000x reference)
import jax
import jax.numpy as jnp
from jax import lax
from jax.experimental import pallas as pl
from jax.experimental.pallas import tpu as pltpu
from jax.experimental.pallas import tpu_sc as plsc

F32 = jnp.float32
I32 = jnp.int32

_NC = 2
_NS = 16
_L = 16
_ROW = 128
_CROWS = 6

_BNP = 1000

_SC_PARAMS = pltpu.CompilerParams(
    use_tc_tiling_on_sc=False, needs_layout_passes=False)


def _mesh():
    return plsc.VectorSubcoreMesh(core_axis_name="c", subcore_axis_name="s")


def _npad(n):
    sl = (n // (_NC * _NS) + 16) // 16 * 16
    return sl, sl * _NC * _NS


def _deg_body(dst_hbm, out_hbm, deg_v, chunk_v):
    c = lax.axis_index("c")
    t = lax.axis_index("s")
    w = c * _NS + t
    n_pad = deg_v.shape[0]
    e_tile = dst_hbm.shape[0] // (_NC * _NS)
    ch = chunk_v.shape[0]
    zeros = jnp.zeros((_L,), F32)
    ones = jnp.ones((_L,), F32)

    def zbody(i, carry):
        deg_v[pl.ds(i * _L, _L)] = zeros
        return carry

    lax.fori_loop(0, n_pad // _L, zbody, 0, unroll=8)

    base = w * e_tile
    for k in range(e_tile // ch):
        pltpu.sync_copy(dst_hbm.at[pl.ds(base + k * ch, ch)], chunk_v)

        def ebody(j, carry):
            idx = chunk_v[pl.ds(j * _L, _L)]
            plsc.addupdate_scatter(deg_v, [idx], ones)
            return carry

        lax.fori_loop(0, ch // _L, ebody, 0, unroll=8)

    pltpu.sync_copy(deg_v, out_hbm.at[w])


def _deg_call(dst_p, n):
    e_pad = dst_p.shape[0]
    _, n_pad = _npad(n)
    out_type = jax.ShapeDtypeStruct((_NC * _NS, n_pad), F32)
    scratch = [
        pltpu.VMEM((n_pad,), F32),
        pltpu.VMEM((e_pad // (_NC * _NS * 8),), I32),
    ]
    return pl.kernel(
        _deg_body, out_type=out_type, mesh=_mesh(), scratch_types=scratch,
        compiler_params=_SC_PARAMS,
    )(dst_p)


def _degsum_body(dp_hbm, out_hbm, abuf, sbuf, rep):
    c = lax.axis_index("c")
    t = lax.axis_index("s")
    w = c * _NS + t
    sl = abuf.shape[0]
    base = w * sl
    pltpu.sync_copy(dp_hbm.at[0, pl.ds(base, sl)], abuf)
    for k in range(1, _NC * _NS):
        pltpu.sync_copy(dp_hbm.at[k, pl.ds(base, sl)], sbuf)

        def rbody(i, carry):
            abuf[pl.ds(i * _L, _L)] = (abuf[pl.ds(i * _L, _L)]
                                       + sbuf[pl.ds(i * _L, _L)])
            return carry

        lax.fori_loop(0, sl // _L, rbody, 0, unroll=8)

    def repl(i, carry):
        iv = jnp.zeros((_L,), I32) + i
        v = plsc.load_gather(abuf, [iv])
        rep[pl.ds(i * 32, _L)] = v
        rep[pl.ds(i * 32 + _L, _L)] = v
        return carry

    lax.fori_loop(0, sl, repl, 0, unroll=4)
    pltpu.sync_copy(rep, out_hbm.at[pl.ds(base * 32, sl * 32)])


def _degsum_call(dp, n):
    sl, n_pad = _npad(n)
    out_type = jax.ShapeDtypeStruct((n_pad * 32,), F32)
    scratch = [
        pltpu.VMEM((sl,), F32),
        pltpu.VMEM((sl,), F32),
        pltpu.VMEM((sl * 32,), F32),
    ]
    return pl.kernel(
        _degsum_body, out_type=out_type, mesh=_mesh(), scratch_types=scratch,
        compiler_params=_SC_PARAMS,
    )(dp)


def _prop_body(hs_hbm, edges_hbm, out_hbm,
               ebuf_a, ebuf_b, rbuf_a, rbuf_b, isem, gsem, ssem, acc):
    c = lax.axis_index("c")
    t = lax.axis_index("s")
    n = out_hbm.shape[0]
    rows_tile = n // _NS
    zrows = 250
    z = jnp.zeros((_L,), F32)

    def zb(i, carry):
        rbuf_a[i, 0, :] = z
        return carry

    lax.fori_loop(0, zrows, zb, 0, unroll=8)

    row0 = t * rows_tile
    zd = [
        pltpu.async_copy(rbuf_a.at[pl.ds(0, zrows)],
                         acc.at[pl.ds(row0 + k * zrows, zrows)], gsem)
        for k in range(rows_tile // zrows)
    ]
    for d in zd:
        d.wait()
    plsc.subcore_barrier()

    erows_tile = edges_hbm.shape[0] // 2 // _NS
    nbody = erows_tile // (2 * _CROWS)
    rbase0 = t * erows_tile
    cvec = jnp.full((_L,), 0, I32) + c

    def fix_src(ebuf):
        for k in range(_CROWS):
            for v in range(_ROW // _L):
                ebuf[2 * k, pl.ds(v * _L, _L)] = (
                    ebuf[2 * k, pl.ds(v * _L, _L)] + cvec)

    def body(i, carry):
        ca = rbase0 + i * 2 * _CROWS
        cb = ca + _CROWS

        @pl.when(i == 0)
        def _():
            pltpu.async_copy(edges_hbm.at[pl.ds(2 * ca, 2 * _CROWS)],
                             ebuf_a, isem)

        pltpu.make_async_copy(edges_hbm.at[pl.ds(2 * ca, 2 * _CROWS)],
                              ebuf_a, isem).wait()
        fix_src(ebuf_a)
        gda = [
            pltpu.async_copy(hs_hbm.at[ebuf_a.at[2 * k]],
                             rbuf_a.at[pl.ds(k * _ROW, _ROW)], gsem)
            for k in range(_CROWS)
        ]
        db = pltpu.async_copy(edges_hbm.at[pl.ds(2 * cb, 2 * _CROWS)],
                              ebuf_b, isem)
        for d in gda:
            d.wait()
        sda = [
            pltpu.async_copy(rbuf_a.at[pl.ds(k * _ROW, _ROW)],
                             acc.at[ebuf_a.at[2 * k + 1]], ssem, add=True)
            for k in range(_CROWS)
        ]
        db.wait()
        fix_src(ebuf_b)
        gdb = [
            pltpu.async_copy(hs_hbm.at[ebuf_b.at[2 * k]],
                             rbuf_b.at[pl.ds(k * _ROW, _ROW)], gsem)
            for k in range(_CROWS)
        ]
        for d in sda:
            d.wait()

        @pl.when(i < nbody - 1)
        def _():
            pltpu.async_copy(
                edges_hbm.at[pl.ds(2 * (ca + 2 * _CROWS), 2 * _CROWS)],
                ebuf_a, isem)

        for d in gdb:
            d.wait()
        sdb = [
            pltpu.async_copy(rbuf_b.at[pl.ds(k * _ROW, _ROW)],
                             acc.at[ebuf_b.at[2 * k + 1]], ssem, add=True)
            for k in range(_CROWS)
        ]
        for d in sdb:
            d.wait()
        return carry

    lax.fori_loop(0, nbody, body, 0)
    plsc.subcore_barrier()
    pltpu.sync_copy(acc.at[pl.ds(row0, rows_tile)],
                    out_hbm.at[pl.ds(row0, rows_tile), pl.ds(c, 1)])


def _prop_call(hs3, edges, n):
    out_type = jax.ShapeDtypeStruct((n, _NC, _L), F32)
    ch = _CROWS * _ROW
    scratch = [
        pltpu.VMEM((2 * _CROWS, _ROW), I32),
        pltpu.VMEM((2 * _CROWS, _ROW), I32),
        pltpu.VMEM((ch, 1, _L), F32),
        pltpu.VMEM((ch, 1, _L), F32),
        pltpu.SemaphoreType.DMA,
        pltpu.SemaphoreType.DMA,
        pltpu.SemaphoreType.DMA,
        pltpu.VMEM_SHARED((n + _L, 1, _L), F32),
    ]
    return pl.kernel(
        _prop_body, out_type=out_type, mesh=_mesh(), scratch_types=scratch,
        compiler_params=pltpu.CompilerParams(
            use_tc_tiling_on_sc=False, needs_layout_passes=False,
            internal_scratch_in_bytes=131072),
    )(hs3, edges)


def _tc_first(xp, degp, bd_w1, n4):
    def body(x_ref, dg_ref, w_ref, hs_ref, dinv_ref):
        deg = dg_ref[...]
        dinv = jnp.where(deg > 0, lax.rsqrt(jnp.maximum(deg, 1.0)), 0.0)
        h = jnp.dot(x_ref[...], w_ref[...], preferred_element_type=F32)
        hs_ref[...] = h * dinv
        dinv_ref[...] = dinv

    return pl.pallas_call(
        body,
        grid=(n4 // _BNP,),
        in_specs=[
            pl.BlockSpec((_BNP, 20), lambda i: (i, 0)),
            pl.BlockSpec((_BNP, 128), lambda i: (i, 0)),
            pl.BlockSpec((20, 128), lambda i: (0, 0)),
        ],
        out_specs=[
            pl.BlockSpec((_BNP, 128), lambda i: (i, 0)),
            pl.BlockSpec((_BNP, 128), lambda i: (i, 0)),
        ],
        out_shape=[
            jax.ShapeDtypeStruct((n4, 128), F32),
            jax.ShapeDtypeStruct((n4, 128), F32),
        ],
    )(xp, degp, bd_w1)


def _tc_mid(propp, dinvp, bp, bd_w, n4):
    def body(p_ref, di_ref, b_ref, w_ref, hs_ref):
        di = di_ref[...]
        h = jnp.maximum(p_ref[...] * di + b_ref[...], 0.0)
        hs_ref[...] = jnp.dot(
            h, w_ref[...], preferred_element_type=F32) * di

    return pl.pallas_call(
        body,
        grid=(n4 // _BNP,),
        in_specs=[
            pl.BlockSpec((_BNP, 128), lambda i: (i, 0)),
            pl.BlockSpec((_BNP, 128), lambda i: (i, 0)),
            pl.BlockSpec((1, 128), lambda i: (0, 0)),
            pl.BlockSpec((128, 128), lambda i: (0, 0)),
        ],
        out_specs=pl.BlockSpec((_BNP, 128), lambda i: (i, 0)),
        out_shape=jax.ShapeDtypeStruct((n4, 128), F32),
    )(propp, dinvp, bp, bd_w)


def _tc_final(propp, dinvp, b3p, bd_d, bd_c, bdbc, bd_ta, fold, btaf, n4):
    def body(p_ref, di_ref, b_ref, wd_ref, wc_ref, bdc_ref, wta_ref,
             fold_ref, bta_ref,
             dist_ref, chase_ref, tls_ref, al_ref, smax_ref, srow_ref):
        i = pl.program_id(0)
        di = di_ref[...]
        h = jnp.maximum(p_ref[...] * di + b_ref[...], 0.0)
        dist4 = (jnp.dot(h, wd_ref[...], preferred_element_type=F32)
                 + bdc_ref[0, 0])
        chase4 = (jnp.dot(h, wc_ref[...], preferred_element_type=F32)
                  + bdc_ref[0, 1])
        dist_ref[...] = dist4
        chase_ref[...] = chase4

        @pl.when(i == 0)
        def _():
            smax_ref[0] = -jnp.inf

        bm = jnp.max(chase4)

        @pl.when(bm > smax_ref[0])
        def _():
            smax_ref[0] = bm
            ids = (lax.broadcasted_iota(I32, chase4.shape, 0) * 4
                   + lax.broadcasted_iota(I32, chase4.shape, 1)
                   + i * (_BNP * 4))
            amid = jnp.min(jnp.where(chase4 >= bm, ids, jnp.iinfo(I32).max))
            oh = (ids == amid).astype(F32)
            oh128 = jnp.dot(oh, fold_ref[...],
                            preferred_element_type=F32)
            srow_ref[...] = jnp.sum(h * oh128, axis=0, keepdims=True)

        @pl.when(i == pl.num_programs(0) - 1)
        def _():
            ta44 = jnp.dot(srow_ref[...], wta_ref[...],
                           preferred_element_type=F32)
            ta = (ta44[:, 0:11] + ta44[:, 11:22] + ta44[:, 22:33]
                  + ta44[:, 33:44]) + bta_ref[...]
            tls_ref[...] = ta[:, 0:2]
            al_ref[...] = ta[:, 2:11]

    return pl.pallas_call(
        body,
        grid=(n4 // _BNP,),
        in_specs=[
            pl.BlockSpec((_BNP, 128), lambda i: (i, 0)),
            pl.BlockSpec((_BNP, 128), lambda i: (i, 0)),
            pl.BlockSpec((1, 128), lambda i: (0, 0)),
            pl.BlockSpec((128, 4), lambda i: (0, 0)),
            pl.BlockSpec((128, 4), lambda i: (0, 0)),
            pl.BlockSpec((1, 2), lambda i: (0, 0)),
            pl.BlockSpec((128, 44), lambda i: (0, 0)),
            pl.BlockSpec((4, 128), lambda i: (0, 0)),
            pl.BlockSpec((1, 11), lambda i: (0, 0)),
        ],
        out_specs=[
            pl.BlockSpec((_BNP, 4), lambda i: (i, 0)),
            pl.BlockSpec((_BNP, 4), lambda i: (i, 0)),
            pl.BlockSpec((1, 2), lambda i: (0, 0)),
            pl.BlockSpec((1, 9), lambda i: (0, 0)),
        ],
        out_shape=[
            jax.ShapeDtypeStruct((n4, 4), F32),
            jax.ShapeDtypeStruct((n4, 4), F32),
            jax.ShapeDtypeStruct((1, 2), F32),
            jax.ShapeDtypeStruct((1, 9), F32),
        ],
        scratch_shapes=[
            pltpu.SMEM((1,), F32),
            pltpu.VMEM((1, 128), F32),
        ],
    )(propp, dinvp, b3p, bd_d, bd_c, bdbc, bd_ta, fold, btaf)


def kernel(x, edge_index, W1, b1, W2, b2, W3, b3,
           Wd, bd, Wc, bc, Wt, bt, Wa, ba):
    n = x.shape[0]
    n4 = n // 4
    e = edge_index.shape[1]
    unit = _NS * _ROW * _CROWS * 2
    e_pad = (e + unit - 1) // unit * unit
    pad = e_pad - e

    src = edge_index[0]
    dst = edge_index[1]
    src2 = jnp.concatenate([src * 2, jnp.zeros((pad,), I32)])
    dst_p = jnp.concatenate([dst, jnp.full((pad,), n, I32)])
    edges = jnp.stack(
        [src2.reshape(-1, _ROW), dst_p.reshape(-1, _ROW)],
        axis=1).reshape(-1, _ROW)

    eye4 = jnp.eye(4, dtype=F32)
    dpart = _deg_call(dst_p, n)
    degrep = _degsum_call(dpart, n)
    degp = degrep[:n * 32].reshape(n4, 128)

    xp = x.reshape(n4, 20)
    hs1p, dinvp = _tc_first(xp, degp, jnp.kron(eye4, W1), n4)

    def prop(hsp):
        hs3 = hsp.reshape(2 * n, 1, _L)
        return _prop_call(hs3, edges, n).reshape(n4, 128)

    p1 = prop(hs1p)
    hs2p = _tc_mid(p1, dinvp, jnp.tile(b1, 4).reshape(1, 128),
                   jnp.kron(eye4, W2), n4)
    p2 = prop(hs2p)
    hs3p = _tc_mid(p2, dinvp, jnp.tile(b2, 4).reshape(1, 128),
                   jnp.kron(eye4, W3), n4)
    p3 = prop(hs3p)

    wta = jnp.concatenate([Wt, Wa], axis=1)
    bta = jnp.concatenate([bt, ba]).reshape(1, 11)
    fold = jnp.kron(eye4, jnp.ones((1, 32), F32))
    dist4, chase4, tls, al = _tc_final(
        p3, dinvp, jnp.tile(b3, 4).reshape(1, 128),
        jnp.kron(eye4, Wd), jnp.kron(eye4, Wc),
        jnp.stack([bd, bc], axis=1), jnp.kron(eye4, wta), fold, bta, n4)
    return (dist4.reshape(n), chase4.reshape(n),
            tls.reshape(2), al.reshape(9))


# speedup vs baseline: 3.3606x; 3.3606x over previous
import jax
import jax.numpy as jnp
from jax import lax
from jax.experimental import pallas as pl
from jax.experimental.pallas import tpu as pltpu
from jax.experimental.pallas import tpu_sc as plsc

F32 = jnp.float32
I32 = jnp.int32

_NC = 2
_NS = 16
_L = 16
_ROW = 128
_CROWS = 6

_BNR = 1280

_SC_PARAMS = pltpu.CompilerParams(
    use_tc_tiling_on_sc=False, needs_layout_passes=False)


def _mesh():
    return plsc.VectorSubcoreMesh(core_axis_name="c", subcore_axis_name="s")


def _npad(n):
    sl = (n // (_NC * _NS) + 16) // 16 * 16
    return sl, sl * _NC * _NS


def _deg_body(dst_hbm, out_hbm, deg_v, chunk_v):
    c = lax.axis_index("c")
    t = lax.axis_index("s")
    w = c * _NS + t
    n_pad = deg_v.shape[0]
    e_tile = dst_hbm.shape[0] // (_NC * _NS)
    ch = chunk_v.shape[0]
    zeros = jnp.zeros((_L,), F32)
    ones = jnp.ones((_L,), F32)

    def zbody(i, carry):
        deg_v[pl.ds(i * _L, _L)] = zeros
        return carry

    lax.fori_loop(0, n_pad // _L, zbody, 0, unroll=8)

    base = w * e_tile
    for k in range(e_tile // ch):
        pltpu.sync_copy(dst_hbm.at[pl.ds(base + k * ch, ch)], chunk_v)

        def ebody(j, carry):
            idx = chunk_v[pl.ds(j * _L, _L)]
            plsc.addupdate_scatter(deg_v, [idx], ones)
            return carry

        lax.fori_loop(0, ch // _L, ebody, 0, unroll=8)

    pltpu.sync_copy(deg_v, out_hbm.at[w])


def _deg_call(dst_p, n):
    e_pad = dst_p.shape[0]
    _, n_pad = _npad(n)
    out_type = jax.ShapeDtypeStruct((_NC * _NS, n_pad), F32)
    scratch = [
        pltpu.VMEM((n_pad,), F32),
        pltpu.VMEM((e_pad // (_NC * _NS * 8),), I32),
    ]
    return pl.kernel(
        _deg_body, out_type=out_type, mesh=_mesh(), scratch_types=scratch,
        compiler_params=_SC_PARAMS,
    )(dst_p)


def _degsum_body(dp_hbm, out_hbm, abuf, sbuf, rep):
    c = lax.axis_index("c")
    t = lax.axis_index("s")
    w = c * _NS + t
    sl = abuf.shape[0]
    base = w * sl
    pltpu.sync_copy(dp_hbm.at[0, pl.ds(base, sl)], abuf)
    for k in range(1, _NC * _NS):
        pltpu.sync_copy(dp_hbm.at[k, pl.ds(base, sl)], sbuf)

        def rbody(i, carry):
            abuf[pl.ds(i * _L, _L)] = (abuf[pl.ds(i * _L, _L)]
                                       + sbuf[pl.ds(i * _L, _L)])
            return carry

        lax.fori_loop(0, sl // _L, rbody, 0, unroll=8)

    def repl(i, carry):
        iv = jnp.zeros((_L,), I32) + i
        rep[pl.ds(i * _L, _L)] = plsc.load_gather(abuf, [iv])
        return carry

    lax.fori_loop(0, sl, repl, 0, unroll=4)
    pltpu.sync_copy(rep, out_hbm.at[pl.ds(base * _L, sl * _L)])


def _degsum_call(dp, n):
    sl, n_pad = _npad(n)
    out_type = jax.ShapeDtypeStruct((n_pad * _L,), F32)
    scratch = [
        pltpu.VMEM((sl,), F32),
        pltpu.VMEM((sl,), F32),
        pltpu.VMEM((sl * _L,), F32),
    ]
    return pl.kernel(
        _degsum_body, out_type=out_type, mesh=_mesh(), scratch_types=scratch,
        compiler_params=_SC_PARAMS,
    )(dp)


def _prop_body(hs_hbm, edges_hbm, out_hbm,
               ebuf_a, ebuf_b, rbuf_a, rbuf_b, isem, gsem, ssem, acc):
    c = lax.axis_index("c")
    t = lax.axis_index("s")
    n = out_hbm.shape[1]
    rows_tile = n // _NS
    zrows = 256
    z = jnp.zeros((_L,), F32)

    def zb(i, carry):
        rbuf_a[i, :] = z
        return carry

    lax.fori_loop(0, zrows, zb, 0, unroll=8)

    row0 = t * rows_tile
    zd = [
        pltpu.async_copy(rbuf_a.at[pl.ds(0, zrows)],
                         acc.at[pl.ds(row0 + k * zrows, zrows)], gsem)
        for k in range(rows_tile // zrows)
    ]
    for d in zd:
        d.wait()
    plsc.subcore_barrier()

    tbl = hs_hbm.at[c]
    erows_tile = edges_hbm.shape[0] // 2 // _NS
    nbody = erows_tile // (2 * _CROWS)
    rbase0 = t * erows_tile

    def body(i, carry):
        ca = rbase0 + i * 2 * _CROWS
        cb = ca + _CROWS

        @pl.when(i == 0)
        def _():
            pltpu.async_copy(edges_hbm.at[pl.ds(2 * ca, 2 * _CROWS)],
                             ebuf_a, isem)

        pltpu.make_async_copy(edges_hbm.at[pl.ds(2 * ca, 2 * _CROWS)],
                              ebuf_a, isem).wait()
        gda = [
            pltpu.async_copy(tbl.at[ebuf_a.at[2 * k]],
                             rbuf_a.at[pl.ds(k * _ROW, _ROW)], gsem)
            for k in range(_CROWS)
        ]
        @pl.when(i > 0)
        def _():
            for k in range(_CROWS):
                pltpu.make_async_copy(
                    rbuf_b.at[pl.ds(k * _ROW, _ROW)],
                    acc.at[ebuf_b.at[2 * k + 1]], ssem).wait()

        db = pltpu.async_copy(edges_hbm.at[pl.ds(2 * cb, 2 * _CROWS)],
                              ebuf_b, isem)
        for d in gda:
            d.wait()
        sda = [
            pltpu.async_copy(rbuf_a.at[pl.ds(k * _ROW, _ROW)],
                             acc.at[ebuf_a.at[2 * k + 1]], ssem, add=True)
            for k in range(_CROWS)
        ]
        db.wait()
        gdb = [
            pltpu.async_copy(tbl.at[ebuf_b.at[2 * k]],
                             rbuf_b.at[pl.ds(k * _ROW, _ROW)], gsem)
            for k in range(_CROWS)
        ]
        for d in sda:
            d.wait()

        @pl.when(i < nbody - 1)
        def _():
            pltpu.async_copy(
                edges_hbm.at[pl.ds(2 * (ca + 2 * _CROWS), 2 * _CROWS)],
                ebuf_a, isem)

        for d in gdb:
            d.wait()
        for k in range(_CROWS):
            pltpu.async_copy(rbuf_b.at[pl.ds(k * _ROW, _ROW)],
                             acc.at[ebuf_b.at[2 * k + 1]], ssem, add=True)
        return carry

    lax.fori_loop(0, nbody, body, 0)
    for k in range(_CROWS):
        pltpu.make_async_copy(rbuf_b.at[pl.ds(k * _ROW, _ROW)],
                              acc.at[ebuf_b.at[2 * k + 1]], ssem).wait()
    plsc.subcore_barrier()
    pltpu.sync_copy(acc.at[pl.ds(row0, rows_tile)],
                    out_hbm.at[c, pl.ds(row0, rows_tile)])


def _prop_call(hs, edges, n):
    out_type = jax.ShapeDtypeStruct((_NC, n, _L), F32)
    ch = _CROWS * _ROW
    scratch = [
        pltpu.VMEM((2 * _CROWS, _ROW), I32),
        pltpu.VMEM((2 * _CROWS, _ROW), I32),
        pltpu.VMEM((ch, _L), F32),
        pltpu.VMEM((ch, _L), F32),
        pltpu.SemaphoreType.DMA,
        pltpu.SemaphoreType.DMA,
        pltpu.SemaphoreType.DMA,
        pltpu.VMEM_SHARED((n + _L, _L), F32),
    ]
    return pl.kernel(
        _prop_body, out_type=out_type, mesh=_mesh(), scratch_types=scratch,
        compiler_params=pltpu.CompilerParams(
            use_tc_tiling_on_sc=False, needs_layout_passes=False,
            internal_scratch_in_bytes=131072),
    )(hs, edges)


def _tc_first(xp8, degh, w1k, n8):
    def body(x_ref, dg_ref, w_ref, hs_ref, dinv_ref):
        deg = dg_ref[...]
        dinv = jnp.where(deg > 0, lax.rsqrt(jnp.maximum(deg, 1.0)), 0.0)
        h = jnp.dot(x_ref[...], w_ref[0], preferred_element_type=F32)
        hs_ref[...] = (h * dinv)[None]
        dinv_ref[...] = dinv

    return pl.pallas_call(
        body,
        grid=(_NC, n8 // _BNR),
        in_specs=[
            pl.BlockSpec((_BNR, 40), lambda c, i: (i, 0)),
            pl.BlockSpec((_BNR, 128), lambda c, i: (i, 0)),
            pl.BlockSpec((1, 40, 128), lambda c, i: (c, 0, 0)),
        ],
        out_specs=[
            pl.BlockSpec((1, _BNR, 128), lambda c, i: (c, i, 0)),
            pl.BlockSpec((_BNR, 128), lambda c, i: (i, 0)),
        ],
        out_shape=[
            jax.ShapeDtypeStruct((_NC, n8, 128), F32),
            jax.ShapeDtypeStruct((n8, 128), F32),
        ],
    )(xp8, degh, w1k)


def _tc_mid(prop8, dinvh, bk, wk, n8):
    def body(p0_ref, p1_ref, di_ref, b_ref, w_ref, hs_ref):
        di = di_ref[...]
        h0 = jnp.maximum(p0_ref[0] * di + b_ref[0, 0], 0.0)
        h1 = jnp.maximum(p1_ref[0] * di + b_ref[0, 1], 0.0)
        acc = (jnp.dot(h0, w_ref[0, 0], preferred_element_type=F32)
               + jnp.dot(h1, w_ref[0, 1], preferred_element_type=F32))
        hs_ref[...] = (acc * di)[None]

    return pl.pallas_call(
        body,
        grid=(_NC, n8 // _BNR),
        in_specs=[
            pl.BlockSpec((1, _BNR, 128), lambda c, i: (0, i, 0)),
            pl.BlockSpec((1, _BNR, 128), lambda c, i: (1, i, 0)),
            pl.BlockSpec((_BNR, 128), lambda c, i: (i, 0)),
            pl.BlockSpec((1, 2, 128), lambda c, i: (0, 0, 0)),
            pl.BlockSpec((1, 2, 128, 128), lambda c, i: (c, 0, 0, 0)),
        ],
        out_specs=pl.BlockSpec((1, _BNR, 128), lambda c, i: (c, i, 0)),
        out_shape=jax.ShapeDtypeStruct((_NC, n8, 128), F32),
    )(prop8, prop8, dinvh, bk, wk)


def _tc_final(prop8, dinvh, b3k, wdk, wck, bdc, wtak, fold8, btaf, n8,
              nreal):
    def body(p0_ref, p1_ref, di_ref, b_ref, wd_ref, wc_ref, bdc_ref,
             wta_ref, fold_ref, bta_ref,
             dist_ref, chase_ref, tls_ref, al_ref,
             smax_ref, srow0_ref, srow1_ref):
        i = pl.program_id(0)
        di = di_ref[...]
        h0 = jnp.maximum(p0_ref[0] * di + b_ref[0, 0], 0.0)
        h1 = jnp.maximum(p1_ref[0] * di + b_ref[0, 1], 0.0)
        dist8 = (jnp.dot(h0, wd_ref[0], preferred_element_type=F32)
                 + jnp.dot(h1, wd_ref[1], preferred_element_type=F32)
                 + bdc_ref[0, 0])
        chase8 = (jnp.dot(h0, wc_ref[0], preferred_element_type=F32)
                  + jnp.dot(h1, wc_ref[1], preferred_element_type=F32)
                  + bdc_ref[0, 1])
        dist_ref[...] = dist8
        chase_ref[...] = chase8

        @pl.when(i == 0)
        def _():
            smax_ref[0] = -jnp.inf

        ids = (lax.broadcasted_iota(I32, chase8.shape, 0) * 8
               + lax.broadcasted_iota(I32, chase8.shape, 1)
               + i * (_BNR * 8))
        chase8m = jnp.where(ids < nreal, chase8, -jnp.inf)
        bm = jnp.max(chase8m)

        @pl.when(bm > smax_ref[0])
        def _():
            smax_ref[0] = bm
            amid = jnp.min(jnp.where(chase8m >= bm, ids,
                                     jnp.iinfo(I32).max))
            oh = (ids == amid).astype(F32)
            oh128 = jnp.dot(oh, fold_ref[...],
                            preferred_element_type=F32)
            srow0_ref[...] = jnp.sum(h0 * oh128, axis=0, keepdims=True)
            srow1_ref[...] = jnp.sum(h1 * oh128, axis=0, keepdims=True)

        @pl.when(i == pl.num_programs(0) - 1)
        def _():
            ta88 = (jnp.dot(srow0_ref[...], wta_ref[0],
                            preferred_element_type=F32)
                    + jnp.dot(srow1_ref[...], wta_ref[1],
                              preferred_element_type=F32))
            ta = bta_ref[...]
            for m in range(8):
                ta = ta + ta88[:, 11 * m:11 * m + 11]
            tls_ref[...] = ta[:, 0:2]
            al_ref[...] = ta[:, 2:11]

    return pl.pallas_call(
        body,
        grid=(n8 // _BNR,),
        in_specs=[
            pl.BlockSpec((1, _BNR, 128), lambda i: (0, i, 0)),
            pl.BlockSpec((1, _BNR, 128), lambda i: (1, i, 0)),
            pl.BlockSpec((_BNR, 128), lambda i: (i, 0)),
            pl.BlockSpec((1, 2, 128), lambda i: (0, 0, 0)),
            pl.BlockSpec((2, 128, 8), lambda i: (0, 0, 0)),
            pl.BlockSpec((2, 128, 8), lambda i: (0, 0, 0)),
            pl.BlockSpec((1, 2), lambda i: (0, 0)),
            pl.BlockSpec((2, 128, 88), lambda i: (0, 0, 0)),
            pl.BlockSpec((8, 128), lambda i: (0, 0)),
            pl.BlockSpec((1, 11), lambda i: (0, 0)),
        ],
        out_specs=[
            pl.BlockSpec((_BNR, 8), lambda i: (i, 0)),
            pl.BlockSpec((_BNR, 8), lambda i: (i, 0)),
            pl.BlockSpec((1, 2), lambda i: (0, 0)),
            pl.BlockSpec((1, 9), lambda i: (0, 0)),
        ],
        out_shape=[
            jax.ShapeDtypeStruct((n8, 8), F32),
            jax.ShapeDtypeStruct((n8, 8), F32),
            jax.ShapeDtypeStruct((1, 2), F32),
            jax.ShapeDtypeStruct((1, 9), F32),
        ],
        scratch_shapes=[
            pltpu.SMEM((1,), F32),
            pltpu.VMEM((1, 128), F32),
            pltpu.VMEM((1, 128), F32),
        ],
    )(prop8, prop8, dinvh, b3k, wdk, wck, bdc, wtak, fold8, btaf)


def kernel(x, edge_index, W1, b1, W2, b2, W3, b3,
           Wd, bd, Wc, bc, Wt, bt, Wa, ba):
    n = x.shape[0]
    n8 = n // 8
    n8p = (n8 + _BNR - 1) // _BNR * _BNR
    n2 = n8p * 8
    e = edge_index.shape[1]
    unit = _NS * _ROW * _CROWS * 2
    e_pad = (e + unit - 1) // unit * unit
    pad = e_pad - e

    src = edge_index[0]
    dst = edge_index[1]
    src_p = jnp.concatenate([src, jnp.zeros((pad,), I32)])
    dst_p = jnp.concatenate([dst, jnp.full((pad,), n, I32)])
    edges = jnp.stack(
        [src_p.reshape(-1, _ROW), dst_p.reshape(-1, _ROW)],
        axis=1).reshape(-1, _ROW)

    eye8 = jnp.eye(8, dtype=F32)
    dpart = _deg_call(dst_p, n)
    degrep = _degsum_call(dpart, n)
    degh = jnp.concatenate(
        [degrep[:n * _L],
         jnp.zeros(((n2 - n) * _L,), F32)]).reshape(n8p, 128)

    def bd8(m):
        return jnp.kron(eye8, m)

    xp8 = jnp.concatenate(
        [x.reshape(n8, 40), jnp.zeros((n8p - n8, 40), F32)])
    w1k = jnp.stack([bd8(W1[:, :16]), bd8(W1[:, 16:])])
    hs1, dinvh = _tc_first(xp8, degh, w1k, n8p)

    def wk(w):
        return jnp.stack([
            jnp.stack([bd8(w[:16, :16]), bd8(w[16:, :16])]),
            jnp.stack([bd8(w[:16, 16:]), bd8(w[16:, 16:])]),
        ])

    def bk(b):
        return jnp.stack([jnp.tile(b[:16], 8), jnp.tile(b[16:], 8)])[None]

    def prop(hsp):
        hs = hsp.reshape(_NC, n2, _L)
        return _prop_call(hs, edges, n2).reshape(_NC, n8p, 128)

    p1 = prop(hs1)
    hs2 = _tc_mid(p1, dinvh, bk(b1), wk(W2), n8p)
    p2 = prop(hs2)
    hs3 = _tc_mid(p2, dinvh, bk(b2), wk(W3), n8p)
    p3 = prop(hs3)

    wta = jnp.concatenate([Wt, Wa], axis=1)
    wdk = jnp.stack([bd8(Wd[:16]), bd8(Wd[16:])])
    wck = jnp.stack([bd8(Wc[:16]), bd8(Wc[16:])])
    wtak = jnp.stack([bd8(wta[:16]), bd8(wta[16:])])
    fold8 = jnp.kron(eye8, jnp.ones((1, 16), F32))
    dist8, chase8, tls, al = _tc_final(
        p3, dinvh, bk(b3), wdk, wck, jnp.stack([bd, bc], axis=1),
        wtak, fold8, jnp.concatenate([bt, ba]).reshape(1, 11), n8p, n)
    return (dist8.reshape(n2)[:n], chase8.reshape(n2)[:n],
            tls.reshape(2), al.reshape(9))
